# baseline (speedup 1.0000x reference)
import functools

import jax
import jax.numpy as jnp
from jax import lax
from jax.experimental import pallas as pl
from jax.experimental.pallas import tpu as pltpu
from jax.experimental.pallas import tpu_sc as plsc

_info = plsc.get_sparse_core_info()
_NC, _NS = _info.num_cores, _info.num_subcores
_NW = _NC * _NS
_CH = 128


@functools.lru_cache(maxsize=None)
def _make_gather(batch: int, dim2: int):
    b_per_w = batch // _NW
    n_chunks = b_per_w // _CH
    half = n_chunks // 2
    mesh = plsc.VectorSubcoreMesh(core_axis_name="c", subcore_axis_name="s")

    @functools.partial(
        pl.kernel,
        mesh=mesh,
        out_type=jax.ShapeDtypeStruct((_NW, b_per_w, dim2), jnp.float32),
        scratch_types=[
            pltpu.VMEM((n_chunks, _CH), jnp.int32),
            pltpu.VMEM((2, _CH, dim2), jnp.float32),
            pltpu.SemaphoreType.DMA,
            pltpu.SemaphoreType.DMA,
        ],
    )
    def gather_kernel(table_hbm, pidx_hbm, out_hbm, pidx_v, pairs_v,
                      sem0, sem1):
        wid = lax.axis_index("s") * _NC + lax.axis_index("c")
        sems = (sem0, sem1)
        pltpu.sync_copy(pidx_hbm.at[wid], pidx_v)
        pltpu.async_copy(table_hbm.at[pidx_v.at[0]], pairs_v.at[0], sem0)
        pltpu.async_copy(table_hbm.at[pidx_v.at[1]], pairs_v.at[1], sem1)

        def body(i, _):
            for b in range(2):
                j = 2 * i + b
                pltpu.make_async_copy(
                    table_hbm.at[pl.ds(0, _CH)], pairs_v.at[b], sems[b]
                ).wait()
                pltpu.sync_copy(
                    pairs_v.at[b], out_hbm.at[wid, pl.ds(j * _CH, _CH)]
                )

                @pl.when(i < half - 1)
                def _():
                    pltpu.async_copy(
                        table_hbm.at[pidx_v.at[j + 2]], pairs_v.at[b], sems[b]
                    )
            return ()

        lax.fori_loop(0, half, body, ())

    return gather_kernel


def kernel(states, embedding):
    batch = states.shape[0]
    v, dim = embedding.shape
    idx = states.astype(jnp.int32).reshape(batch)
    t2 = embedding.reshape(v // 2, 2 * dim)
    pidx = (idx >> 1).reshape(_NW, batch // (_NW * _CH), _CH)
    pairs = _make_gather(batch, 2 * dim)(t2, pidx)
    pairs = pairs.reshape(batch, 2, dim)
    return jnp.take_along_axis(pairs, (idx & 1)[:, None, None], axis=1)[:, 0]
